# baseline (speedup 1.0000x reference)
import functools

import jax
import jax.numpy as jnp
from jax import lax
from jax.experimental import pallas as pl
from jax.experimental.pallas import tpu as pltpu
from jax.experimental.pallas import tpu_sc as plsc

VOCAB = 1000000
EMBED_DIM = 64
BATCH = 4096
HIST = 200
TOT = BATCH * HIST

_info = plsc.get_sparse_core_info()
_NCORES = _info.num_cores
_NSUB = _info.num_subcores
NW = _NCORES * _NSUB
PER = TOT // NW
CHUNK = 128
NSTEPS = PER // CHUNK

_mesh = plsc.VectorSubcoreMesh(core_axis_name="c", subcore_axis_name="s")


@functools.partial(
    pl.kernel,
    mesh=_mesh,
    out_type=jax.ShapeDtypeStruct((TOT, EMBED_DIM), jnp.float32),
    scratch_types=[
        pltpu.VMEM((1, CHUNK), jnp.int32),
        pltpu.VMEM((CHUNK, EMBED_DIM), jnp.float32),
        pltpu.SemaphoreType.DMA,
    ],
    compiler_params=pltpu.CompilerParams(use_tc_tiling_on_sc=False),
)
def _sc_gather(idx_hbm, table_hbm, out_hbm, idx_v, rows_v, sem):
    wid = lax.axis_index("s") * _NCORES + lax.axis_index("c")
    base = wid * PER

    def step(i, carry):
        off = base + i * CHUNK
        pltpu.sync_copy(idx_hbm.at[pl.ds(off, CHUNK)], idx_v.at[0])
        pltpu.async_copy(table_hbm.at[idx_v.at[0]], rows_v, sem).wait()
        pltpu.sync_copy(rows_v, out_hbm.at[pl.ds(off, CHUNK)])
        return carry

    lax.fori_loop(0, NSTEPS, step, 0)


def kernel(input, table):
    idx = input.reshape(TOT).astype(jnp.int32)
    out = _sc_gather(idx, table)
    return out.reshape(BATCH, HIST, EMBED_DIM)


# speedup vs baseline: 1.1821x; 1.1821x over previous
import functools

import jax
import jax.numpy as jnp
from jax import lax
from jax.experimental import pallas as pl
from jax.experimental.pallas import tpu as pltpu
from jax.experimental.pallas import tpu_sc as plsc

VOCAB = 1000000
EMBED_DIM = 64
BATCH = 4096
HIST = 200
TOT = BATCH * HIST

_info = plsc.get_sparse_core_info()
_NCORES = _info.num_cores
_NSUB = _info.num_subcores
NW = _NCORES * _NSUB
PER = TOT // NW
CHUNK = 128
NSTEPS = PER // CHUNK
NBUF = 4
LOOKAHEAD = 2

_mesh = plsc.VectorSubcoreMesh(core_axis_name="c", subcore_axis_name="s")


@functools.partial(
    pl.kernel,
    mesh=_mesh,
    out_type=jax.ShapeDtypeStruct((TOT, EMBED_DIM), jnp.float32),
    scratch_types=[
        pltpu.VMEM((PER,), jnp.int32),
        pltpu.VMEM((NBUF, CHUNK, EMBED_DIM), jnp.float32),
        pltpu.SemaphoreType.DMA((NBUF,)),
        pltpu.SemaphoreType.DMA((NBUF,)),
        pltpu.SemaphoreType.DMA,
    ],
    compiler_params=pltpu.CompilerParams(use_tc_tiling_on_sc=False),
)
def _sc_gather(idx_hbm, table_hbm, out_hbm, idx_v, rows_v, gsem, ssem, isem):
    wid = lax.axis_index("s") * _NCORES + lax.axis_index("c")
    base = wid * PER

    pltpu.async_copy(idx_hbm.at[pl.ds(base, PER)], idx_v, isem).wait()

    def issue_gather(k, slot):
        pltpu.async_copy(
            table_hbm.at[idx_v.at[pl.ds(k * CHUNK, CHUNK)]],
            rows_v.at[slot],
            gsem.at[slot],
        )

    def gather_wait(slot):
        pltpu.make_async_copy(
            table_hbm.at[pl.ds(0, CHUNK)], rows_v.at[slot], gsem.at[slot]
        ).wait()

    def issue_store(k, slot):
        pltpu.async_copy(
            rows_v.at[slot], out_hbm.at[pl.ds(base + k * CHUNK, CHUNK)],
            ssem.at[slot],
        )

    def store_wait(slot):
        pltpu.make_async_copy(
            table_hbm.at[pl.ds(0, CHUNK)], rows_v.at[slot], ssem.at[slot]
        ).wait()

    for k in range(LOOKAHEAD):
        issue_gather(k, k % NBUF)
    for k in range(LOOKAHEAD):
        slot = k % NBUF
        gather_wait(slot)
        issue_store(k, slot)
        issue_gather(k + LOOKAHEAD, (k + LOOKAHEAD) % NBUF)

    n_main = NSTEPS - 2 * LOOKAHEAD
    assert n_main % NBUF == 0

    def outer(m, carry):
        k0 = LOOKAHEAD + m * NBUF
        for b in range(NBUF):
            slot = (LOOKAHEAD + b) % NBUF
            k = k0 + b
            gather_wait(slot)
            issue_store(k, slot)
            nslot = (LOOKAHEAD + b + LOOKAHEAD) % NBUF
            store_wait(nslot)
            issue_gather(k + LOOKAHEAD, nslot)
        return carry

    lax.fori_loop(0, n_main // NBUF, outer, 0)

    for k in range(NSTEPS - LOOKAHEAD, NSTEPS):
        slot = k % NBUF
        gather_wait(slot)
        issue_store(k, slot)

    for b in range(NBUF):
        store_wait(b)


def kernel(input, table):
    idx = input.reshape(TOT).astype(jnp.int32)
    out = _sc_gather(idx, table)
    return out.reshape(BATCH, HIST, EMBED_DIM)


# speedup vs baseline: 1.4531x; 1.2292x over previous
import functools

import jax
import jax.numpy as jnp
from jax import lax
from jax.experimental import pallas as pl
from jax.experimental.pallas import tpu as pltpu
from jax.experimental.pallas import tpu_sc as plsc

VOCAB = 1000000
EMBED_DIM = 64
WIDE = 128
BATCH = 4096
HIST = 200
TOT = BATCH * HIST

_info = plsc.get_sparse_core_info()
_NCORES = _info.num_cores
_NSUB = _info.num_subcores
NW = _NCORES * _NSUB
PER = TOT // NW
CHUNK = 128
NSTEPS = PER // CHUNK
NBUF = 4
LOOKAHEAD = 2

_mesh = plsc.VectorSubcoreMesh(core_axis_name="c", subcore_axis_name="s")


@functools.partial(
    pl.kernel,
    mesh=_mesh,
    out_type=jax.ShapeDtypeStruct((TOT, WIDE), jnp.float32),
    scratch_types=[
        pltpu.VMEM((PER,), jnp.int32),
        pltpu.VMEM((NBUF, CHUNK, WIDE), jnp.float32),
        pltpu.SemaphoreType.DMA((NBUF,)),
        pltpu.SemaphoreType.DMA((NBUF,)),
        pltpu.SemaphoreType.DMA,
    ],
)
def _sc_gather(idx_hbm, table_hbm, out_hbm, idx_v, rows_v, gsem, ssem, isem):
    wid = lax.axis_index("s") * _NCORES + lax.axis_index("c")
    base = wid * PER

    pltpu.async_copy(idx_hbm.at[pl.ds(base, PER)], idx_v, isem).wait()

    def issue_gather(k, slot):
        pltpu.async_copy(
            table_hbm.at[idx_v.at[pl.ds(k * CHUNK, CHUNK)]],
            rows_v.at[slot],
            gsem.at[slot],
        )

    def gather_wait(slot):
        pltpu.make_async_copy(
            table_hbm.at[pl.ds(0, CHUNK)], rows_v.at[slot], gsem.at[slot]
        ).wait()

    def issue_store(k, slot):
        pltpu.async_copy(
            rows_v.at[slot], out_hbm.at[pl.ds(base + k * CHUNK, CHUNK)],
            ssem.at[slot],
        )

    def store_wait(slot):
        pltpu.make_async_copy(
            table_hbm.at[pl.ds(0, CHUNK)], rows_v.at[slot], ssem.at[slot]
        ).wait()

    for k in range(LOOKAHEAD):
        issue_gather(k, k % NBUF)
    for k in range(LOOKAHEAD):
        slot = k % NBUF
        gather_wait(slot)
        issue_store(k, slot)
        issue_gather(k + LOOKAHEAD, (k + LOOKAHEAD) % NBUF)

    n_main = NSTEPS - 2 * LOOKAHEAD
    assert n_main % NBUF == 0

    def outer(m, carry):
        k0 = LOOKAHEAD + m * NBUF
        for b in range(NBUF):
            slot = (LOOKAHEAD + b) % NBUF
            k = k0 + b
            gather_wait(slot)
            issue_store(k, slot)
            nslot = b
            store_wait(nslot)
            issue_gather(k + LOOKAHEAD, nslot)
        return carry

    lax.fori_loop(0, n_main // NBUF, outer, 0)

    for k in range(NSTEPS - LOOKAHEAD, NSTEPS):
        slot = k % NBUF
        gather_wait(slot)
        issue_store(k, slot)

    for b in range(NBUF):
        store_wait(b)


def kernel(input, table):
    idx = input.reshape(TOT).astype(jnp.int32)
    wide = jnp.concatenate(
        [table, jnp.zeros((VOCAB, WIDE - EMBED_DIM), jnp.float32)], axis=1
    )
    out = _sc_gather(idx, wide)
    return out[:, :EMBED_DIM].reshape(BATCH, HIST, EMBED_DIM)
